# Initial kernel scaffold; baseline (speedup 1.0000x reference)
import jax
import jax.numpy as jnp
from jax.experimental import pallas as pl


def kernel(vid_feat, vid_mask, txt_feat, txt_mask, gt, Wq, Wk, Wv, Wo, Ws, We):
    raise NotImplementedError("write your pallas kernel here")


# speedup vs baseline: 4.1239x; 4.1239x over previous
import jax
import jax.numpy as jnp
import numpy as np
from jax.experimental import pallas as pl
from jax.experimental.pallas import tpu as pltpu

B = 16
T = 128
L = 32
D = 512
TOPK = 512
IOU_THR = 0.5
NEG = -1e9


def _dense_body(vid_ref, txt_ref, vmask_ref, tmask_ref, eg_ref, sg_ref,
                wq_ref, wk_ref, wv_ref, wo_ref, ws_ref, we_ref,
                score_ref, gtd_ref):
    inv_sqrt_d = np.float32(1.0 / np.sqrt(D))
    vf = vid_ref[0]
    tf = txt_ref[0]
    vm = vmask_ref[0, 0]
    tm = tmask_ref[0, 0]

    f32 = jnp.float32
    q = jax.lax.dot(vf, wq_ref[...], preferred_element_type=f32)
    k = jax.lax.dot(tf, wk_ref[...], preferred_element_type=f32)
    v = jax.lax.dot(tf, wv_ref[...], preferred_element_type=f32)
    logits = jax.lax.dot_general(q, k, (((1,), (1,)), ((), ())),
                                 preferred_element_type=f32) * inv_sqrt_d
    logits = jnp.where(tm[None, :] > 0, logits, NEG)
    mx = jnp.max(logits, axis=-1, keepdims=True)
    p = jnp.exp(logits - mx)
    attn = p / jnp.sum(p, axis=-1, keepdims=True)
    ctx = jax.lax.dot(attn, v, preferred_element_type=f32)
    vid2 = (vf + jax.lax.dot(ctx, wo_ref[...], preferred_element_type=f32)) \
        * vm[:, None]

    tsum = jnp.sum(tf * tm[:, None], axis=0)
    tpool = tsum / jnp.maximum(jnp.sum(tm), 1.0)
    sfeat = jax.lax.dot(vid2, ws_ref[...], preferred_element_type=f32) \
        * tpool[None, :]
    efeat = jax.lax.dot(vid2, we_ref[...], preferred_element_type=f32)
    s2 = jax.lax.dot_general(sfeat, efeat, (((1,), (1,)), ((), ())),
                             preferred_element_type=f32) * inv_sqrt_d
    ii = jax.lax.broadcasted_iota(jnp.int32, (T, T), 0)
    jj = jax.lax.broadcasted_iota(jnp.int32, (T, T), 1)
    s2 = jnp.where(jj >= ii, s2, NEG)
    score_ref[0] = s2

    t = jax.lax.broadcasted_iota(jnp.int32, (T, 3), 0).astype(f32)
    eg = eg_ref[0, 0]
    sg = sg_ref[0, 0]
    gtd_ref[0] = jnp.exp(-(t - eg[None, :]) ** 2 / (2.0 * sg[None, :] ** 2))


def _nms_body(idx_ref, sc_ref, si_ref, ei_ref, out_ref):
    f32 = jnp.float32
    idx = idx_ref[0, 0, :]
    qq = idx // T
    rr = idx - qq * T
    si = qq.astype(f32) / T
    ei = (rr.astype(f32) + 1.0) / T
    ln = ei - si
    inter = jnp.clip(jnp.minimum(ei[:, None], ei[None, :])
                     - jnp.maximum(si[:, None], si[None, :]), 0.0)
    union = ln[:, None] + ln[None, :] - inter
    iou = inter / jnp.maximum(union, 1e-6)
    ii = jax.lax.broadcasted_iota(jnp.int32, (TOPK, TOPK), 0)
    jj = jax.lax.broadcasted_iota(jnp.int32, (TOPK, TOPK), 1)
    a_f = ((iou > IOU_THR) & (jj > ii)).astype(f32)

    def cond(c):
        return c[1] > 0

    def body(c):
        kv, _ = c
        supp = jax.lax.dot_general(kv, a_f, (((1,), (0,)), ((), ())),
                                   preferred_element_type=f32)
        knew = jnp.where(supp > 0.0, 0.0, 1.0)
        ch = jnp.any(knew != kv).astype(jnp.int32)
        return knew, ch

    k0 = jnp.ones((1, TOPK), dtype=f32)
    kfin, _ = jax.lax.while_loop(cond, body, (k0, jnp.int32(1)))
    keep = kfin[0]
    sc = sc_ref[0, 0, :]
    si_ref[0, 0] = si
    ei_ref[0, 0] = ei
    out_ref[0, 0] = jnp.where(keep > 0, sc, 0.0)


def kernel(vid_feat, vid_mask, txt_feat, txt_mask, gt, Wq, Wk, Wv, Wo, Ws, We):
    f32 = jnp.float32
    mid = (gt[:, 0] + gt[:, 1]) / 2.0
    expanded = jnp.concatenate([gt, mid[:, None]], axis=1)
    eg = T * expanded
    alpha = jnp.array([0.25, 0.25, 0.21], dtype=f32)
    sg = alpha[None, :] * (eg[..., 1] - eg[..., 0])[:, None]

    wspec = pl.BlockSpec((D, D), lambda b: (0, 0))
    score2d, gt_dist = pl.pallas_call(
        _dense_body,
        grid=(B,),
        in_specs=[
            pl.BlockSpec((1, T, D), lambda b: (b, 0, 0)),
            pl.BlockSpec((1, L, D), lambda b: (b, 0, 0)),
            pl.BlockSpec((1, 1, T), lambda b: (b, 0, 0)),
            pl.BlockSpec((1, 1, L), lambda b: (b, 0, 0)),
            pl.BlockSpec((1, 1, 3), lambda b: (b, 0, 0)),
            pl.BlockSpec((1, 1, 3), lambda b: (b, 0, 0)),
            wspec, wspec, wspec, wspec, wspec, wspec,
        ],
        out_specs=[
            pl.BlockSpec((1, T, T), lambda b: (b, 0, 0)),
            pl.BlockSpec((1, T, 3), lambda b: (b, 0, 0)),
        ],
        out_shape=[
            jax.ShapeDtypeStruct((B, T, T), f32),
            jax.ShapeDtypeStruct((B, T, 3), f32),
        ],
    )(vid_feat, txt_feat, vid_mask.reshape(B, 1, T), txt_mask.reshape(B, 1, L),
      eg.reshape(B, 1, 3), sg.reshape(B, 1, 3), Wq, Wk, Wv, Wo, Ws, We)

    flat = score2d.reshape(B, T * T)
    score_raw, top_idx = jax.lax.top_k(flat, TOPK)

    si, ei, nms_score = pl.pallas_call(
        _nms_body,
        grid=(B,),
        in_specs=[
            pl.BlockSpec((1, 1, TOPK), lambda b: (b, 0, 0)),
            pl.BlockSpec((1, 1, TOPK), lambda b: (b, 0, 0)),
        ],
        out_specs=[
            pl.BlockSpec((1, 1, TOPK), lambda b: (b, 0, 0)),
            pl.BlockSpec((1, 1, TOPK), lambda b: (b, 0, 0)),
            pl.BlockSpec((1, 1, TOPK), lambda b: (b, 0, 0)),
        ],
        out_shape=[
            jax.ShapeDtypeStruct((B, 1, TOPK), f32),
            jax.ShapeDtypeStruct((B, 1, TOPK), f32),
            jax.ShapeDtypeStruct((B, 1, TOPK), f32),
        ],
    )(top_idx.reshape(B, 1, TOPK), score_raw.reshape(B, 1, TOPK))

    si = si.reshape(B, TOPK)
    ei = ei.reshape(B, TOPK)
    nms_score = nms_score.reshape(B, TOPK)
    pred_bds = jnp.stack([si, ei], axis=-1)
    return pred_bds, nms_score, gt_dist
